# baseline (speedup 1.0000x reference)
import functools

import jax
import jax.numpy as jnp
from jax import lax
from jax.experimental import pallas as pl
from jax.experimental.pallas import tpu as pltpu
from jax.experimental.pallas import tpu_sc as plsc

BATCH = 16384
SRC_LEN = 200
PART_DIM = 64
OUT_DIM = 16

NC = 2
NS = 16
NW = NC * NS
BW = BATCH // NW
NL = 4
NBLK = SRC_LEN // NL
GROUPS = BW // 16


def _body(src_hbm, w_hbm, b_hbm, q_hbm, out_hbm,
          w_v, b_v, q_v, pc_v, x0, x1, o0, o1, si0, si1, so0, so1):
    wid = lax.axis_index("s") * NC + lax.axis_index("c")
    b0 = wid * BW

    pltpu.sync_copy(w_hbm, w_v)
    pltpu.sync_copy(b_hbm, b_v)
    pltpu.sync_copy(q_hbm, q_v)

    colbase = lax.iota(jnp.int32, 16) * PART_DIM

    def fold_row(i, _):
        wrow = w_v[i, :]
        brow = b_v[i, :]
        acc_w = jnp.zeros((16,), jnp.float32)
        acc_b = jnp.zeros((16,), jnp.float32)
        for k in range(OUT_DIM):
            qrow = q_v[k, :]
            acc_w = acc_w + wrow[k] * qrow
            acc_b = acc_b + brow[k] * qrow
        wbits = plsc.bitcast(acc_w, jnp.int32) & jnp.int32(-65536)
        bbits = ((plsc.bitcast(acc_b, jnp.int32) + 32768) >> 16) & jnp.int32(65535)
        plsc.store_scatter(pc_v, [colbase + i], wbits | bbits)
        return 0

    lax.fori_loop(0, PART_DIM, fold_row, 0)

    def in_copy(blk, x_v, sem):
        return pltpu.make_async_copy(
            src_hbm.at[pl.ds(blk * NL, NL), pl.ds(b0, BW)], x_v, sem)

    def out_copy(blk, o_v, sem):
        return pltpu.make_async_copy(
            o_v, out_hbm.at[pl.ds(blk * NL * OUT_DIM, NL * OUT_DIM),
                            pl.ds(b0, BW)], sem)

    def compute(x_v, o_v):
        @plsc.parallel_loop(0, NL * GROUPS, unroll=1)
        def do_group(g):
            l = g // GROUPS
            g16 = g % GROUPS
            x = x_v[l, pl.ds(g16 * 16, 16)]
            xc = jnp.minimum(x, 64.0)
            t = xc.astype(jnp.int32)
            tf = t.astype(jnp.float32)
            idx = jnp.clip(jnp.where(xc > tf, t + 1, t), 0, PART_DIM - 1)
            row0 = l * OUT_DIM
            for c in range(OUT_DIM):
                gv = plsc.load_gather(pc_v.at[pl.ds(c * PART_DIM, PART_DIM)],
                                      [idx])
                w = plsc.bitcast(gv, jnp.float32)
                b = plsc.bitcast(gv << 16, jnp.float32)
                o_v[row0 + c, pl.ds(g16 * 16, 16)] = x * w + b

    in_copy(0, x0, si0).start()
    in_copy(1, x1, si1).start()

    in_copy(0, x0, si0).wait()
    compute(x0, o0)
    out_copy(0, o0, so0).start()
    in_copy(2, x0, si0).start()

    in_copy(1, x1, si1).wait()
    compute(x1, o1)
    out_copy(1, o1, so1).start()
    in_copy(3, x1, si1).start()

    def steady(p, _):
        blk0 = 2 * p + 2
        in_copy(blk0, x0, si0).wait()
        out_copy(blk0, o0, so0).wait()
        compute(x0, o0)
        out_copy(blk0, o0, so0).start()
        in_copy(jnp.minimum(blk0 + 2, NBLK - 2), x0, si0).start()

        blk1 = 2 * p + 3
        in_copy(blk1, x1, si1).wait()
        out_copy(blk1, o1, so1).wait()
        compute(x1, o1)
        out_copy(blk1, o1, so1).start()
        in_copy(jnp.minimum(blk1 + 2, NBLK - 1), x1, si1).start()
        return 0

    lax.fori_loop(0, (NBLK - 2) // 2, steady, 0)

    in_copy(0, x0, si0).wait()
    in_copy(1, x1, si1).wait()
    out_copy(0, o0, so0).wait()
    out_copy(1, o1, so1).wait()


@jax.jit
def _run(src_t, weight, bias, quantify):
    mesh = plsc.VectorSubcoreMesh(core_axis_name="c", subcore_axis_name="s")
    f = functools.partial(
        pl.kernel,
        out_type=jax.ShapeDtypeStruct((SRC_LEN * OUT_DIM, BATCH), jnp.float32),
        mesh=mesh,
        compiler_params=pltpu.CompilerParams(needs_layout_passes=False),
        scratch_types=[
            pltpu.VMEM((PART_DIM, OUT_DIM), jnp.float32),
            pltpu.VMEM((PART_DIM, OUT_DIM), jnp.float32),
            pltpu.VMEM((OUT_DIM, OUT_DIM), jnp.float32),
            pltpu.VMEM((PART_DIM * OUT_DIM,), jnp.int32),
            pltpu.VMEM((NL, BW), jnp.float32),
            pltpu.VMEM((NL, BW), jnp.float32),
            pltpu.VMEM((NL * OUT_DIM, BW), jnp.float32),
            pltpu.VMEM((NL * OUT_DIM, BW), jnp.float32),
            pltpu.SemaphoreType.DMA,
            pltpu.SemaphoreType.DMA,
            pltpu.SemaphoreType.DMA,
            pltpu.SemaphoreType.DMA,
        ],
    )(_body)
    return f(src_t, weight, bias, quantify)


def kernel(src, weight, bias, quantify, proj_parts):
    del proj_parts
    out2 = _run(src.T, weight, bias, quantify)
    return out2.reshape(SRC_LEN, OUT_DIM, BATCH).transpose(2, 0, 1)


# speedup vs baseline: 1.0375x; 1.0375x over previous
import functools

import jax
import jax.numpy as jnp
from jax import lax
from jax.experimental import pallas as pl
from jax.experimental.pallas import tpu as pltpu
from jax.experimental.pallas import tpu_sc as plsc

BATCH = 16384
SRC_LEN = 200
PART_DIM = 64
OUT_DIM = 16

NC = 2
NS = 16
NW = NC * NS
BSLOTS = 16
BW = BATCH // BSLOTS
LSPAN = SRC_LEN // 2
NL = 2
NBLK = LSPAN // NL
GROUPS = BW // 16


def _body(src_hbm, w_hbm, b_hbm, q_hbm, out_hbm,
          w_v, b_v, q_v, pc_v, x0, x1, o0, o1, si0, si1, so0, so1):
    wid = lax.axis_index("s") * NC + lax.axis_index("c")
    b0 = (wid % BSLOTS) * BW
    l0 = (wid // BSLOTS) * LSPAN

    pltpu.sync_copy(w_hbm, w_v)
    pltpu.sync_copy(b_hbm, b_v)
    pltpu.sync_copy(q_hbm, q_v)

    colbase = lax.iota(jnp.int32, 16) * PART_DIM

    def fold_row(i, _):
        wrow = w_v[i, :]
        brow = b_v[i, :]
        acc_w = jnp.zeros((16,), jnp.float32)
        acc_b = jnp.zeros((16,), jnp.float32)
        for k in range(OUT_DIM):
            qrow = q_v[k, :]
            acc_w = acc_w + wrow[k] * qrow
            acc_b = acc_b + brow[k] * qrow
        wbits = plsc.bitcast(acc_w, jnp.int32) & jnp.int32(-65536)
        bbits = ((plsc.bitcast(acc_b, jnp.int32) + 32768) >> 16) & jnp.int32(65535)
        plsc.store_scatter(pc_v, [colbase + i], wbits | bbits)
        return 0

    lax.fori_loop(0, PART_DIM, fold_row, 0)

    def in_copy(blk, x_v, sem):
        return pltpu.make_async_copy(
            src_hbm.at[pl.ds(l0 + blk * NL, NL), pl.ds(b0, BW)], x_v, sem)

    def out_copy(blk, o_v, sem):
        return pltpu.make_async_copy(
            o_v, out_hbm.at[pl.ds((l0 + blk * NL) * OUT_DIM, NL * OUT_DIM),
                            pl.ds(b0, BW)], sem)

    def compute(x_v, o_v):
        @plsc.parallel_loop(0, NL * GROUPS, unroll=2)
        def do_group(g):
            l = g // GROUPS
            g16 = g % GROUPS
            x = x_v[l, pl.ds(g16 * 16, 16)]
            xc = jnp.minimum(x, 64.0)
            t = xc.astype(jnp.int32)
            tf = t.astype(jnp.float32)
            idx = jnp.clip(jnp.where(xc > tf, t + 1, t), 0, PART_DIM - 1)
            row0 = l * OUT_DIM
            for c in range(OUT_DIM):
                gv = plsc.load_gather(pc_v.at[pl.ds(c * PART_DIM, PART_DIM)],
                                      [idx])
                w = plsc.bitcast(gv, jnp.float32)
                b = plsc.bitcast(gv << 16, jnp.float32)
                o_v[row0 + c, pl.ds(g16 * 16, 16)] = x * w + b

    in_copy(0, x0, si0).start()
    in_copy(1, x1, si1).start()

    in_copy(0, x0, si0).wait()
    compute(x0, o0)
    out_copy(0, o0, so0).start()
    in_copy(2, x0, si0).start()

    in_copy(1, x1, si1).wait()
    compute(x1, o1)
    out_copy(1, o1, so1).start()
    in_copy(3, x1, si1).start()

    def steady(p, _):
        blk0 = 2 * p + 2
        in_copy(blk0, x0, si0).wait()
        out_copy(blk0, o0, so0).wait()
        compute(x0, o0)
        out_copy(blk0, o0, so0).start()
        in_copy(jnp.minimum(blk0 + 2, NBLK - 2), x0, si0).start()

        blk1 = 2 * p + 3
        in_copy(blk1, x1, si1).wait()
        out_copy(blk1, o1, so1).wait()
        compute(x1, o1)
        out_copy(blk1, o1, so1).start()
        in_copy(jnp.minimum(blk1 + 2, NBLK - 1), x1, si1).start()
        return 0

    lax.fori_loop(0, (NBLK - 2) // 2, steady, 0)

    in_copy(0, x0, si0).wait()
    in_copy(1, x1, si1).wait()
    out_copy(0, o0, so0).wait()
    out_copy(1, o1, so1).wait()


@jax.jit
def _run(src_t, weight, bias, quantify):
    mesh = plsc.VectorSubcoreMesh(core_axis_name="c", subcore_axis_name="s")
    f = functools.partial(
        pl.kernel,
        out_type=jax.ShapeDtypeStruct((SRC_LEN * OUT_DIM, BATCH), jnp.float32),
        mesh=mesh,
        compiler_params=pltpu.CompilerParams(needs_layout_passes=False),
        scratch_types=[
            pltpu.VMEM((PART_DIM, OUT_DIM), jnp.float32),
            pltpu.VMEM((PART_DIM, OUT_DIM), jnp.float32),
            pltpu.VMEM((OUT_DIM, OUT_DIM), jnp.float32),
            pltpu.VMEM((PART_DIM * OUT_DIM,), jnp.int32),
            pltpu.VMEM((NL, BW), jnp.float32),
            pltpu.VMEM((NL, BW), jnp.float32),
            pltpu.VMEM((NL * OUT_DIM, BW), jnp.float32),
            pltpu.VMEM((NL * OUT_DIM, BW), jnp.float32),
            pltpu.SemaphoreType.DMA,
            pltpu.SemaphoreType.DMA,
            pltpu.SemaphoreType.DMA,
            pltpu.SemaphoreType.DMA,
        ],
    )(_body)
    return f(src_t, weight, bias, quantify)


def kernel(src, weight, bias, quantify, proj_parts):
    del proj_parts
    out2 = _run(src.T, weight, bias, quantify)
    return out2.reshape(SRC_LEN, OUT_DIM, BATCH).transpose(2, 0, 1)


# speedup vs baseline: 1.0531x; 1.0150x over previous
import functools

import jax
import jax.numpy as jnp
from jax import lax
from jax.experimental import pallas as pl
from jax.experimental.pallas import tpu as pltpu
from jax.experimental.pallas import tpu_sc as plsc

BATCH = 16384
SRC_LEN = 200
PART_DIM = 64
OUT_DIM = 16

NC = 2
NS = 16
NW = NC * NS
BSLOTS = 8
BW = BATCH // BSLOTS
LSPAN = SRC_LEN // 4
NL = 1
NBLK = LSPAN // NL
GROUPS = BW // 16


def _body(src_hbm, w_hbm, b_hbm, q_hbm, out_hbm,
          w_v, b_v, q_v, pc_v, x0, x1, o0, o1, si0, si1, so0, so1):
    wid = lax.axis_index("s") * NC + lax.axis_index("c")
    b0 = (wid % BSLOTS) * BW
    l0 = (wid // BSLOTS) * LSPAN

    pltpu.sync_copy(w_hbm, w_v)
    pltpu.sync_copy(b_hbm, b_v)
    pltpu.sync_copy(q_hbm, q_v)

    colbase = lax.iota(jnp.int32, 16) * PART_DIM

    def fold_row(i, _):
        wrow = w_v[i, :]
        brow = b_v[i, :]
        acc_w = jnp.zeros((16,), jnp.float32)
        acc_b = jnp.zeros((16,), jnp.float32)
        for k in range(OUT_DIM):
            qrow = q_v[k, :]
            acc_w = acc_w + wrow[k] * qrow
            acc_b = acc_b + brow[k] * qrow
        wbits = plsc.bitcast(acc_w, jnp.int32) & jnp.int32(-65536)
        bbits = ((plsc.bitcast(acc_b, jnp.int32) + 32768) >> 16) & jnp.int32(65535)
        plsc.store_scatter(pc_v, [colbase + i], wbits | bbits)
        return 0

    lax.fori_loop(0, PART_DIM, fold_row, 0)

    def in_copy(blk, x_v, sem):
        return pltpu.make_async_copy(
            src_hbm.at[pl.ds(l0 + blk * NL, NL), pl.ds(b0, BW)], x_v, sem)

    def out_copy(blk, o_v, sem):
        return pltpu.make_async_copy(
            o_v, out_hbm.at[pl.ds((l0 + blk * NL) * OUT_DIM, NL * OUT_DIM),
                            pl.ds(b0, BW)], sem)

    def compute(x_v, o_v):
        @plsc.parallel_loop(0, NL * GROUPS, unroll=2)
        def do_group(g):
            l = g // GROUPS
            g16 = g % GROUPS
            x = x_v[l, pl.ds(g16 * 16, 16)]
            xc = jnp.minimum(x, 64.0)
            t = xc.astype(jnp.int32)
            tf = t.astype(jnp.float32)
            idx = jnp.clip(jnp.where(xc > tf, t + 1, t), 0, PART_DIM - 1)
            row0 = l * OUT_DIM
            for c in range(OUT_DIM):
                gv = plsc.load_gather(pc_v.at[pl.ds(c * PART_DIM, PART_DIM)],
                                      [idx])
                w = plsc.bitcast(gv, jnp.float32)
                b = plsc.bitcast(gv << 16, jnp.float32)
                o_v[row0 + c, pl.ds(g16 * 16, 16)] = x * w + b

    in_copy(0, x0, si0).start()
    in_copy(1, x1, si1).start()

    in_copy(0, x0, si0).wait()
    compute(x0, o0)
    out_copy(0, o0, so0).start()
    in_copy(2, x0, si0).start()

    in_copy(1, x1, si1).wait()
    compute(x1, o1)
    out_copy(1, o1, so1).start()
    in_copy(3, x1, si1).start()

    def steady(p, _):
        blk0 = 2 * p + 2
        in_copy(blk0, x0, si0).wait()
        out_copy(blk0, o0, so0).wait()
        compute(x0, o0)
        out_copy(blk0, o0, so0).start()
        in_copy(jnp.minimum(blk0 + 2, NBLK - 2), x0, si0).start()

        blk1 = 2 * p + 3
        in_copy(blk1, x1, si1).wait()
        out_copy(blk1, o1, so1).wait()
        compute(x1, o1)
        out_copy(blk1, o1, so1).start()
        in_copy(jnp.minimum(blk1 + 2, NBLK - 1), x1, si1).start()
        return 0

    lax.fori_loop(0, (NBLK - 2) // 2, steady, 0)

    in_copy(0, x0, si0).wait()
    in_copy(1, x1, si1).wait()
    out_copy(0, o0, so0).wait()
    out_copy(1, o1, so1).wait()


@jax.jit
def _run(src_t, weight, bias, quantify):
    mesh = plsc.VectorSubcoreMesh(core_axis_name="c", subcore_axis_name="s")
    f = functools.partial(
        pl.kernel,
        out_type=jax.ShapeDtypeStruct((SRC_LEN * OUT_DIM, BATCH), jnp.float32),
        mesh=mesh,
        compiler_params=pltpu.CompilerParams(needs_layout_passes=False),
        scratch_types=[
            pltpu.VMEM((PART_DIM, OUT_DIM), jnp.float32),
            pltpu.VMEM((PART_DIM, OUT_DIM), jnp.float32),
            pltpu.VMEM((OUT_DIM, OUT_DIM), jnp.float32),
            pltpu.VMEM((PART_DIM * OUT_DIM,), jnp.int32),
            pltpu.VMEM((NL, BW), jnp.float32),
            pltpu.VMEM((NL, BW), jnp.float32),
            pltpu.VMEM((NL * OUT_DIM, BW), jnp.float32),
            pltpu.VMEM((NL * OUT_DIM, BW), jnp.float32),
            pltpu.SemaphoreType.DMA,
            pltpu.SemaphoreType.DMA,
            pltpu.SemaphoreType.DMA,
            pltpu.SemaphoreType.DMA,
        ],
    )(_body)
    return f(src_t, weight, bias, quantify)


def kernel(src, weight, bias, quantify, proj_parts):
    del proj_parts
    out2 = _run(src.T, weight, bias, quantify)
    return out2.reshape(SRC_LEN, OUT_DIM, BATCH).transpose(2, 0, 1)
